# baseline (speedup 1.0000x reference)
import functools

import jax
import jax.numpy as jnp
from jax import lax
from jax.experimental import pallas as pl
from jax.experimental.pallas import tpu as pltpu
from jax.experimental.pallas import tpu_sc as plsc

B = 1024
D = 8192
SPLIT = 2
D2 = D // SPLIT
NC = 2
NS = 16
L = 16
NW = NC * NS
BPW = B // NW
NROUND = BPW * SPLIT
NBUF = 8
NVEC = 8

_mesh = plsc.VectorSubcoreMesh(
    core_axis_name="c", subcore_axis_name="s", num_cores=NC, num_subcores=NS)


@functools.partial(
    pl.kernel,
    out_type=jax.ShapeDtypeStruct((NW, L), jnp.float32),
    mesh=_mesh,
    scratch_types=[
        pltpu.VMEM((NROUND, 1), jnp.int32),
        pltpu.VMEM((NBUF, 1, D2), jnp.float32),
        pltpu.VMEM((NBUF, 1, D2), jnp.float32),
        pltpu.VMEM((L,), jnp.float32),
        pltpu.SemaphoreType.DMA((NBUF,)),
        pltpu.SemaphoreType.DMA((NBUF,)),
    ],
)
def _center_loss_partials(feat_hbm, lab_hbm, cent_hbm, out_hbm,
                          idx_v, fbuf, cbuf, accv, fsems, csems):
    wid = lax.axis_index("s") * NC + lax.axis_index("c")
    base = wid * NROUND
    pltpu.sync_copy(lab_hbm.at[wid], idx_v)

    def start(g, b):
        pltpu.make_async_copy(
            feat_hbm.at[pl.ds(base + g, 1)], fbuf.at[b], fsems.at[b]).start()
        pltpu.make_async_copy(
            cent_hbm.at[idx_v.at[g]], cbuf.at[b], csems.at[b]).start()

    def wait(b):
        pltpu.make_async_copy(
            feat_hbm.at[pl.ds(0, 1)], fbuf.at[b], fsems.at[b]).wait()
        pltpu.make_async_copy(
            cent_hbm.at[idx_v.at[0]], cbuf.at[b], csems.at[b]).wait()

    def compute(b, accs):
        def vbody(i, accs):
            f = [fbuf[b, 0, pl.ds(i + j * L, L)] for j in range(NVEC)]
            c = [cbuf[b, 0, pl.ds(i + j * L, L)] for j in range(NVEC)]
            d = [f[j] - c[j] for j in range(NVEC)]
            return tuple(accs[j] + d[j] * d[j] for j in range(NVEC))
        return plsc.parallel_loop(0, D2, step=NVEC * L, carry=accs)(vbody)

    for b in range(NBUF):
        start(b, b)

    def outer(t, accs):
        for b in range(NBUF):
            g = t * NBUF + b
            wait(b)
            accs = compute(b, accs)

            @pl.when(g + NBUF < NROUND)
            def _():
                start(g + NBUF, b)
        return accs

    zero = jnp.zeros((L,), jnp.float32)
    accs = lax.fori_loop(0, NROUND // NBUF, outer, (zero,) * NVEC)
    acc = accs[0]
    for j in range(1, NVEC):
        acc = acc + accs[j]
    accv[...] = acc
    pltpu.sync_copy(accv, out_hbm.at[wid])


def kernel(features, labels, centers):
    lab = labels.astype(jnp.int32).reshape(NW, BPW)
    lab2 = (lab[:, :, None] * SPLIT + jnp.arange(SPLIT, dtype=jnp.int32)
            ).reshape(NW, NROUND, 1)
    feat2 = features.reshape(B * SPLIT, D2)
    cent2 = centers.reshape(centers.shape[0] * SPLIT, D2)
    partials = _center_loss_partials(feat2, lab2, cent2)
    return 0.5 * jnp.sum(partials) / features.shape[0]


# speedup vs baseline: 2.2937x; 2.2937x over previous
import functools

import jax
import jax.numpy as jnp
from jax import lax
from jax.experimental import pallas as pl
from jax.experimental.pallas import tpu as pltpu
from jax.experimental.pallas import tpu_sc as plsc

B = 1024
D = 8192
NC = 2
NS = 16
L = 16
NW = NC * NS
BPW = B // NW
FCH = 4
NFB = 2
NFG = BPW // FCH
NCB = 4
NVEC = 8

_mesh = plsc.VectorSubcoreMesh(
    core_axis_name="c", subcore_axis_name="s", num_cores=NC, num_subcores=NS)


@functools.partial(
    pl.kernel,
    out_type=jax.ShapeDtypeStruct((NW, L), jnp.float32),
    mesh=_mesh,
    scratch_types=[
        pltpu.VMEM((BPW, 1), jnp.int32),
        pltpu.VMEM((NFB, FCH, D), jnp.float32),
        pltpu.VMEM((NCB, 1, D), jnp.float32),
        pltpu.VMEM((L,), jnp.float32),
        pltpu.SemaphoreType.DMA((NFB,)),
        pltpu.SemaphoreType.DMA((NCB,)),
    ],
)
def _center_loss_partials(feat_hbm, lab_hbm, cent_hbm, out_hbm,
                          idx_v, fbuf, cbuf, accv, fsems, csems):
    wid = lax.axis_index("s") * NC + lax.axis_index("c")
    base = wid * BPW
    pltpu.sync_copy(lab_hbm.at[wid], idx_v)

    def fstart(t, fb):
        pltpu.make_async_copy(
            feat_hbm.at[pl.ds(base + t * FCH, FCH)], fbuf.at[fb],
            fsems.at[fb]).start()

    def fwait(fb):
        pltpu.make_async_copy(
            feat_hbm.at[pl.ds(0, FCH)], fbuf.at[fb], fsems.at[fb]).wait()

    def cstart(g, cb):
        pltpu.make_async_copy(
            cent_hbm.at[idx_v.at[g]], cbuf.at[cb], csems.at[cb]).start()

    def cwait(cb):
        pltpu.make_async_copy(
            cent_hbm.at[idx_v.at[0]], cbuf.at[cb], csems.at[cb]).wait()

    def compute(fb, r, cb, accs):
        def vbody(i, accs):
            f = [fbuf[fb, r, pl.ds(i + j * L, L)] for j in range(NVEC)]
            c = [cbuf[cb, 0, pl.ds(i + j * L, L)] for j in range(NVEC)]
            d = [f[j] - c[j] for j in range(NVEC)]
            return tuple(accs[j] + d[j] * d[j] for j in range(NVEC))
        return plsc.parallel_loop(0, D, step=NVEC * L, carry=accs)(vbody)

    for fb in range(NFB):
        fstart(fb, fb)
    for cb in range(NCB):
        cstart(cb, cb)

    def outer(tt, accs):
        for ft in range(NFB):
            t = tt * NFB + ft
            fwait(ft)
            for r in range(FCH):
                g = t * FCH + r
                cb = r
                cwait(cb)
                accs = compute(ft, r, cb, accs)

                @pl.when(g + NCB < BPW)
                def _():
                    cstart(g + NCB, cb)

            @pl.when(t + NFB < NFG)
            def _():
                fstart(t + NFB, ft)
        return accs

    zero = jnp.zeros((L,), jnp.float32)
    accs = lax.fori_loop(0, NFG // NFB, outer, (zero,) * NVEC)
    acc = accs[0]
    for j in range(1, NVEC):
        acc = acc + accs[j]
    accv[...] = acc
    pltpu.sync_copy(accv, out_hbm.at[wid])


def kernel(features, labels, centers):
    lab = labels.astype(jnp.int32).reshape(NW, BPW, 1)
    partials = _center_loss_partials(features, lab, centers)
    return 0.5 * jnp.sum(partials) / features.shape[0]
